# Initial kernel scaffold; baseline (speedup 1.0000x reference)
import jax
import jax.numpy as jnp
from jax.experimental import pallas as pl


def kernel(query, value, reference_points, spatial_shapes, level_start_index, W_off, b_off, W_attn, b_attn, W_val, b_val):
    raise NotImplementedError("write your pallas kernel here")


# speedup vs baseline: 15.1536x; 15.1536x over previous
import jax
import jax.numpy as jnp
from jax import lax
from jax.experimental import pallas as pl
from jax.experimental.pallas import tpu as pltpu
from jax.experimental.pallas import tpu_sc as plsc

BS = 6
NQ = 2500
H_FEAT = 100
W_FEAT = 176
NV = H_FEAT * W_FEAT
D = 256
NH = 8
NP = 8
NZ = 4
DH = D // NH
HP = NH * NP

NCORES = 2
NSUB = 16
NWORK = NCORES * NSUB
LANES = 16

TOT_ROWS = BS * NQ * NH
NQROWS = BS * NQ
ROWS_PER_W = TOT_ROWS // NWORK
CHUNK = 30
NCHUNK = ROWS_PER_W // CHUNK
CPG = CHUNK * 32
REG = CHUNK * NP
CBLK = NQROWS * HP


def _vproj_body(val_ref, wv_ref, bv_ref, out_ref):
    out_ref[0] = (
        jnp.dot(val_ref[0], wv_ref[...], preferred_element_type=jnp.float32)
        + bv_ref[0]
    )


def _vproj(value, w_val, b_val):
    vb = 2200
    return pl.pallas_call(
        _vproj_body,
        grid=(BS, NV // vb),
        in_specs=[
            pl.BlockSpec((1, vb, D), lambda b, i: (b, i, 0)),
            pl.BlockSpec((D, D), lambda b, i: (0, 0)),
            pl.BlockSpec((1, D), lambda b, i: (0, 0)),
        ],
        out_specs=pl.BlockSpec((1, vb, D), lambda b, i: (b, i, 0)),
        out_shape=jax.ShapeDtypeStruct((BS, NV, D), jnp.float32),
    )(value, w_val, b_val.reshape(1, D))


def _locs_body(q_ref, rp_ref, woff_ref, boff_ref, wattn_ref, battn_ref,
               gsum_ref, idx_ref, w_ref):
    qb = q_ref.shape[0]
    row0 = pl.program_id(0) * qb
    q = q_ref[...]
    off = (
        jnp.dot(q, woff_ref[...], preferred_element_type=jnp.float32)
        + boff_ref[0]
    )
    attn = (
        jnp.dot(q, wattn_ref[...], preferred_element_type=jnp.float32)
        + battn_ref[0]
    )
    e = jnp.exp(attn)
    aw = e / jnp.dot(e, gsum_ref[...], preferred_element_type=jnp.float32)

    col = lax.broadcasted_iota(jnp.int32, (qb, 2 * HP), 1)
    src = 2 * ((col % NP) % NZ) + (col >= HP).astype(jnp.int32)
    rpp = jnp.take_along_axis(rp_ref[...], src, axis=1,
                              mode="promise_in_bounds")

    nrm = jnp.where(col < HP, 1.0 / W_FEAT, 1.0 / H_FEAT)
    scl = jnp.where(col < HP, float(W_FEAT), float(H_FEAT))
    xy = (rpp + off * nrm) * scl - 0.5
    x = xy[:, :HP]
    y = xy[:, HP:]
    x0 = jnp.floor(x)
    y0 = jnp.floor(y)
    lx = x - x0
    ly = y - y0
    x0i = x0.astype(jnp.int32)
    y0i = y0.astype(jnp.int32)

    col64 = lax.broadcasted_iota(jnp.int32, (qb, HP), 1)
    b_vec = (row0 + lax.broadcasted_iota(jnp.int32, (qb, HP), 0)) // NQ
    bh = (b_vec * NV) * NH + (col64 >> 3)
    for c4, (dx, dy) in enumerate(((0, 0), (1, 0), (0, 1), (1, 1))):
        cx = x0i + dx
        cy = y0i + dy
        valid = ((cx >= 0) & (cx < W_FEAT) & (cy >= 0) & (cy < H_FEAT))
        wx = lx if dx else (1.0 - lx)
        wy = ly if dy else (1.0 - ly)
        pix = (jnp.clip(cy, 0, H_FEAT - 1) * W_FEAT
               + jnp.clip(cx, 0, W_FEAT - 1))
        idx_ref[c4] = bh + pix * NH
        w_ref[c4] = aw * wx * wy * valid.astype(jnp.float32)


def _locs(query, ref_points, w_off, b_off, w_attn, b_attn):
    qb = 600
    perm = jnp.concatenate([
        jnp.arange(HP, dtype=jnp.int32) * 2,
        jnp.arange(HP, dtype=jnp.int32) * 2 + 1,
    ])
    w_off2 = w_off[:, perm]
    b_off2 = b_off[perm]
    g64 = jnp.arange(HP, dtype=jnp.int32)
    gsum = (g64[:, None] // NP == g64[None, :] // NP).astype(jnp.float32)
    return pl.pallas_call(
        _locs_body,
        grid=(NQROWS // qb,),
        in_specs=[
            pl.BlockSpec((qb, D), lambda i: (i, 0)),
            pl.BlockSpec((qb, NZ * 2), lambda i: (i, 0)),
            pl.BlockSpec((D, 2 * HP), lambda i: (0, 0)),
            pl.BlockSpec((1, 2 * HP), lambda i: (0, 0)),
            pl.BlockSpec((D, HP), lambda i: (0, 0)),
            pl.BlockSpec((1, HP), lambda i: (0, 0)),
            pl.BlockSpec((HP, HP), lambda i: (0, 0)),
        ],
        out_specs=[
            pl.BlockSpec((4, qb, HP), lambda i: (0, i, 0)),
            pl.BlockSpec((4, qb, HP), lambda i: (0, i, 0)),
        ],
        out_shape=[
            jax.ShapeDtypeStruct((4, NQROWS, HP), jnp.int32),
            jax.ShapeDtypeStruct((4, NQROWS, HP), jnp.float32),
        ],
    )(query.reshape(NQROWS, D), ref_points.reshape(NQROWS, NZ * 2),
      w_off2, b_off2.reshape(1, -1), w_attn, b_attn.reshape(1, -1), gsum)


def _lane_bcast(vec, i):
    return jnp.take_along_axis(
        vec, jnp.full((LANES,), i, jnp.int32), axis=0,
        mode="promise_in_bounds")


def _sc_compute_chunk(g_v, w_v, out_v):
    def pair_body(m, _):
        acc = [jnp.zeros((LANES,), jnp.float32) for _ in range(4)]
        for c4 in range(4):
            w16 = w_v[pl.ds(c4 * REG + m * 16, 16)]
            for i in range(16):
                wb = _lane_bcast(w16, i)
                j = c4 * REG + m * 16 + i
                ga = g_v[j, pl.ds(0, 16)]
                gb = g_v[j, pl.ds(16, 16)]
                k = 0 if i < 8 else 2
                acc[k] = acc[k] + wb * ga
                acc[k + 1] = acc[k + 1] + wb * gb
        out_v[pl.ds(m * 64, 16)] = acc[0]
        out_v[pl.ds(m * 64 + 16, 16)] = acc[1]
        out_v[pl.ds(m * 64 + 32, 16)] = acc[2]
        out_v[pl.ds(m * 64 + 48, 16)] = acc[3]
        return 0

    lax.fori_loop(0, CHUNK // 2, pair_body, 0)


def _sc_sample_kernel(vtab, idxf, wf, out, idx_v0, idx_v1, w_v0, w_v1,
                      g_v0, g_v1, out_v0, out_v1,
                      isem0, isem1, wsem0, wsem1, gsem0, gsem1,
                      osem0, osem1):
    wid = lax.axis_index("s") * NCORES + lax.axis_index("c")
    base = wid * ROWS_PER_W

    idx_v = (idx_v0, idx_v1)
    w_v = (w_v0, w_v1)
    g_v = (g_v0, g_v1)
    out_v = (out_v0, out_v1)
    isem = (isem0, isem1)
    wsem = (wsem0, wsem1)
    gsem = (gsem0, gsem1)
    osem = (osem0, osem1)

    def idx_start(c, s):
        for c4 in range(4):
            pltpu.async_copy(
                idxf.at[pl.ds(c4 * CBLK + (base + c * CHUNK) * NP, REG)],
                idx_v[s].at[pl.ds(c4 * REG, REG)], isem[s])

    def w_start(c, s):
        for c4 in range(4):
            pltpu.async_copy(
                wf.at[pl.ds(c4 * CBLK + (base + c * CHUNK) * NP, REG)],
                w_v[s].at[pl.ds(c4 * REG, REG)], wsem[s])

    def gather_start(s):
        pltpu.async_copy(vtab.at[idx_v[s]], g_v[s], gsem[s])

    def idx_wait(s):
        for c4 in range(4):
            pltpu.make_async_copy(idxf.at[pl.ds(0, REG)],
                                  idx_v[s].at[pl.ds(c4 * REG, REG)],
                                  isem[s]).wait()

    def w_wait(s):
        for c4 in range(4):
            pltpu.make_async_copy(wf.at[pl.ds(0, REG)],
                                  w_v[s].at[pl.ds(c4 * REG, REG)],
                                  wsem[s]).wait()

    def gather_wait(s):
        pltpu.make_async_copy(vtab.at[idx_v[s]], g_v[s], gsem[s]).wait()

    def out_start(c, s):
        pltpu.async_copy(out_v[s],
                         out.at[pl.ds((base + c * CHUNK) * 32, CPG)],
                         osem[s])

    def out_wait(s):
        pltpu.make_async_copy(out_v[s], out.at[pl.ds(0, CPG)],
                              osem[s]).wait()

    idx_start(0, 0)
    idx_wait(0)
    gather_start(0)
    idx_start(1, 1)
    w_start(0, 0)

    def loop2(c2, _):
        for par in (0, 1):
            c = c2 * 2 + par
            s0 = par
            s1 = 1 - par

            idx_wait(s1)
            gather_start(s1)
            gather_wait(s0)

            @pl.when(c + 2 < NCHUNK)
            def _():
                idx_start(c + 2, s0)

            w_start(c + 1, s1)
            w_wait(s0)

            @pl.when(c >= 2)
            def _():
                out_wait(s0)

            _sc_compute_chunk(g_v[s0], w_v[s0], out_v[s0])
            out_start(c, s0)
        return 0

    lax.fori_loop(0, (NCHUNK - 1) // 2, loop2, 0)
    gather_wait(0)
    w_wait(0)
    out_wait(0)
    _sc_compute_chunk(g_v[0], w_v[0], out_v[0])
    out_start(NCHUNK - 1, 0)
    out_wait(1)
    out_wait(0)


def _sc_sample(vtab, idxf, wf):
    mesh = plsc.VectorSubcoreMesh(
        core_axis_name="c", subcore_axis_name="s",
        num_cores=NCORES, num_subcores=NSUB)
    f = pl.kernel(
        _sc_sample_kernel,
        out_type=jax.ShapeDtypeStruct((TOT_ROWS * 32,), jnp.float32),
        mesh=mesh,
        compiler_params=pltpu.CompilerParams(use_tc_tiling_on_sc=False),
        scratch_types=[
            pltpu.VMEM((4 * REG,), jnp.int32),
            pltpu.VMEM((4 * REG,), jnp.int32),
            pltpu.VMEM((4 * REG,), jnp.float32),
            pltpu.VMEM((4 * REG,), jnp.float32),
            pltpu.VMEM((CPG, 32), jnp.float32),
            pltpu.VMEM((CPG, 32), jnp.float32),
            pltpu.VMEM((CPG,), jnp.float32),
            pltpu.VMEM((CPG,), jnp.float32),
        ] + [pltpu.SemaphoreType.DMA] * 8,
    )
    return f(vtab, idxf, wf)


def kernel(query, value, reference_points, spatial_shapes, level_start_index,
           W_off, b_off, W_attn, b_attn, W_val, b_val):
    del spatial_shapes, level_start_index
    v = _vproj(value, W_val, b_val)
    idx, w = _locs(query, reference_points, W_off, b_off, W_attn, b_attn)
    vtab = v.reshape(BS * NV * NH, DH)
    out = _sc_sample(vtab,
                     idx.reshape(4 * CBLK),
                     w.reshape(4 * CBLK))
    return out.reshape(BS, NQ, D)
